# baseline (speedup 1.0000x reference)
import functools

import jax
import jax.numpy as jnp
from jax import lax
from jax.experimental import pallas as pl
from jax.experimental.pallas import tpu as pltpu
from jax.experimental.pallas import tpu_sc as plsc

_N = 10000
_E = 320000
_H = 128
_NC = 2
_NS = 16
_NW = _NC * _NS
_K = 128
_NPAD = 10240
_ROWS = _NPAD // _NS
_CH = 80
_EPAD = _CH * _NW * _K


def _sc_mesh():
    return plsc.VectorSubcoreMesh(core_axis_name="c", subcore_axis_name="s")


def _sc_degree_body(srcs, dsts, ones_h, zeros_h, out,
                    src_slab, dst_slab, ones_v, acc, sem):
    c = lax.axis_index("c")
    s = lax.axis_index("s")
    wid = s * _NC + c
    r0 = s * _ROWS

    def zstep(i, carry):
        pltpu.sync_copy(zeros_h, acc.at[pl.ds(r0 + i * _K, _K)])
        return carry

    lax.fori_loop(0, _ROWS // _K, zstep, 0)
    pltpu.sync_copy(ones_h, ones_v)
    pltpu.sync_copy(srcs.at[wid], src_slab)
    pltpu.sync_copy(dsts.at[wid], dst_slab)
    plsc.subcore_barrier()

    def fire_s(j, carry):
        pltpu.async_copy(ones_v, acc.at[src_slab.at[j]], sem, add=True)
        return carry

    def drain(j, carry):
        pltpu.make_async_copy(ones_v, acc.at[src_slab.at[0]], sem).wait()
        return carry

    lax.fori_loop(0, _CH, fire_s, 0)
    lax.fori_loop(0, _CH, drain, 0)
    plsc.subcore_barrier()
    pltpu.sync_copy(acc.at[pl.ds(r0, _ROWS)], out.at[c, 0, pl.ds(r0, _ROWS)])
    lax.fori_loop(0, _ROWS // _K, zstep, 0)
    plsc.subcore_barrier()

    def fire_d(j, carry):
        pltpu.async_copy(ones_v, acc.at[dst_slab.at[j]], sem, add=True)
        return carry

    lax.fori_loop(0, _CH, fire_d, 0)
    lax.fori_loop(0, _CH, drain, 0)
    plsc.subcore_barrier()
    pltpu.sync_copy(acc.at[pl.ds(r0, _ROWS)], out.at[c, 1, pl.ds(r0, _ROWS)])


def _sc_degree(src_slabs, dst_slabs, ones_h, zeros_h):
    kfn = pl.kernel(
        _sc_degree_body,
        mesh=_sc_mesh(),
        out_type=jax.ShapeDtypeStruct((_NC, 2, _NPAD, _H), jnp.float32),
        scratch_types=[
            pltpu.VMEM((_CH, _K), jnp.int32),
            pltpu.VMEM((_CH, _K), jnp.int32),
            pltpu.VMEM((_K, _H), jnp.float32),
            pltpu.VMEM_SHARED((_NPAD, _H), jnp.float32),
            pltpu.SemaphoreType.DMA,
        ],
    )
    return kfn(src_slabs, dst_slabs, ones_h, zeros_h)


_CH0 = 160
_CH1 = 0
_BLK = 40


def _sc_spmm_body(z, srcs, dsts, zeros_h, out,
                  src_slab, dst_slab, rows0, acc, gsem0):
    c = lax.axis_index("c")
    s = lax.axis_index("s")
    r0 = s * _ROWS
    cnt = _CH0 + c * (_CH1 - _CH0)
    start = c * (_NS * _CH0) + s * cnt

    def zstep(i, carry):
        pltpu.sync_copy(zeros_h, acc.at[pl.ds(r0 + i * _K, _K)])
        return carry

    lax.fori_loop(0, _ROWS // _K, zstep, 0)
    plsc.subcore_barrier()

    def blk(b, carry):
        base = start + b * _BLK
        pltpu.sync_copy(srcs.at[pl.ds(base, _BLK)], src_slab)
        pltpu.sync_copy(dsts.at[pl.ds(base, _BLK)], dst_slab)

        def step(j, carry2):
            pltpu.async_copy(z.at[src_slab.at[j]], rows0, gsem0).wait()
            pltpu.sync_copy(rows0, acc.at[dst_slab.at[j]], add=True)
            return carry2

        lax.fori_loop(0, _BLK, step, 0)
        return carry

    lax.fori_loop(0, cnt // _BLK, blk, 0)
    plsc.subcore_barrier()
    pltpu.sync_copy(acc.at[pl.ds(r0, _ROWS)], out.at[c, pl.ds(r0, _ROWS)])


def _sc_spmm(z, src_flat, dst_flat, zeros_h):
    w = z.shape[1]
    kfn = pl.kernel(
        _sc_spmm_body,
        mesh=_sc_mesh(),
        out_type=jax.ShapeDtypeStruct((_NC, _NPAD, w), jnp.float32),
        scratch_types=[
            pltpu.VMEM((_BLK, _K), jnp.int32),
            pltpu.VMEM((_BLK, _K), jnp.int32),
            pltpu.VMEM((_K, w), jnp.float32),
            pltpu.VMEM_SHARED((_NPAD, w), jnp.float32),
            pltpu.SemaphoreType.DMA,
        ],
    )
    return kfn(z, src_flat, dst_flat, zeros_h)


_M = 512


def _norms_from_hist(h4):
    deg_s = h4[0, 0, :, 0:1] + h4[1, 0, :, 0:1]
    deg_d = h4[0, 1, :, 0:1] + h4[1, 1, :, 0:1]
    norm_s = lax.rsqrt(jnp.maximum(deg_s, 1.0))
    norm_d = lax.rsqrt(jnp.maximum(deg_d, 1.0))
    return norm_s, norm_d


def _tc_z0_body(hist_ref, x_ref, w_ref, o_ref):
    norm_s, _ = _norms_from_hist(hist_ref[...])
    o_ref[...] = jnp.dot(x_ref[...] * norm_s, w_ref[...],
                         preferred_element_type=jnp.float32)


def _tc_z0(hist, x, w):
    return pl.pallas_call(
        _tc_z0_body,
        grid=(_NPAD // _M,),
        in_specs=[
            pl.BlockSpec((_NC, 2, _M, _H), lambda i: (0, 0, i, 0)),
            pl.BlockSpec((_M, _H), lambda i: (i, 0)),
            pl.BlockSpec((_H, _H), lambda i: (0, 0)),
        ],
        out_specs=pl.BlockSpec((_M, _H), lambda i: (i, 0)),
        out_shape=jax.ShapeDtypeStruct((_NPAD, _H), jnp.float32),
    )(hist, x, w)


def _tc_mid_body(hist_ref, acc_ref, b_ref, w_ref, o_ref):
    norm_s, norm_d = _norms_from_hist(hist_ref[...])
    a = acc_ref[...]
    y = (a[0] + a[1]) * norm_d + b_ref[...]
    h = jnp.maximum(y, 0.0)
    o_ref[...] = jnp.dot(h * norm_s, w_ref[...],
                         preferred_element_type=jnp.float32)


def _tc_mid(hist, acc, b, w):
    wo = w.shape[1]
    return pl.pallas_call(
        _tc_mid_body,
        grid=(_NPAD // _M,),
        in_specs=[
            pl.BlockSpec((_NC, 2, _M, _H), lambda i: (0, 0, i, 0)),
            pl.BlockSpec((_NC, _M, _H), lambda i: (0, i, 0)),
            pl.BlockSpec((1, _H), lambda i: (0, 0)),
            pl.BlockSpec((_H, wo), lambda i: (0, 0)),
        ],
        out_specs=pl.BlockSpec((_M, wo), lambda i: (i, 0)),
        out_shape=jax.ShapeDtypeStruct((_NPAD, wo), jnp.float32),
    )(hist, acc, b, w)


def _tc_colors_body(hist_ref, acc_ref, b2_ref, colors_ref):
    h4 = hist_ref[...]
    deg_d = h4[0, 1, :, 0:1] + h4[1, 1, :, 0:1]
    norm_d = lax.rsqrt(jnp.maximum(deg_d, 1.0))
    a = acc_ref[...]
    y = (a[0] + a[1]) * norm_d + b2_ref[...]
    neg = jnp.float32(-1e30)
    col = lax.broadcasted_iota(jnp.int32, (_M, _H), 1)
    colmask = col < 3
    logits = jnp.where(colmask, y, neg)
    m = jnp.max(logits, axis=1, keepdims=True)
    e = jnp.where(colmask, jnp.exp(logits - m), 0.0)
    colors = e / jnp.sum(e, axis=1, keepdims=True)
    colors_ref[...] = colors[:, :16]


def _tc_colors(hist, acc, b2):
    return pl.pallas_call(
        _tc_colors_body,
        grid=(_NPAD // _M,),
        in_specs=[
            pl.BlockSpec((_NC, 2, _M, _H), lambda i: (0, 0, i, 0)),
            pl.BlockSpec((_NC, _M, _H), lambda i: (0, i, 0)),
            pl.BlockSpec((1, _H), lambda i: (0, 0)),
        ],
        out_specs=pl.BlockSpec((_M, 16), lambda i: (i, 0)),
        out_shape=jax.ShapeDtypeStruct((_NPAD, 16), jnp.float32),
    )(hist, acc, b2)


def _tc_pool_body(colors_ref, wp_ref, bp_ref, wo_ref, bo_ref, sat_ref):
    colors = colors_ref[...]
    neg = jnp.float32(-1e30)
    gate = jnp.sum(colors * wp_ref[...], axis=1, keepdims=True) + bp_ref[0, 0]
    row = lax.broadcasted_iota(jnp.int32, (_NPAD, 1), 0)
    rowmask = row < _N
    glog = jnp.where(rowmask, gate, neg)
    gm = jnp.max(glog, axis=0, keepdims=True)
    ge = jnp.where(rowmask, jnp.exp(glog - gm), 0.0)
    gw = ge / jnp.sum(ge, axis=0, keepdims=True)
    readout = jnp.sum(gw * colors, axis=0, keepdims=True)
    logit = jnp.sum(readout * wo_ref[...]) + bo_ref[0, 0]
    sat = 1.0 / (1.0 + jnp.exp(-logit))
    sat_ref[...] = jnp.reshape(sat, (1, 1))


def _tc_pool(colors, wp, bp, wo, bo):
    return pl.pallas_call(
        _tc_pool_body,
        out_shape=jax.ShapeDtypeStruct((1, 1), jnp.float32),
    )(colors, wp, bp, wo, bo)


def kernel(x, edge_index, W0, b0, W1, b1, W2, b2, Wp, bp, Wo, bo):
    f32 = jnp.float32
    src = edge_index[0]
    dst = edge_index[1]
    pad = jnp.full((_EPAD - _E,), _N, dtype=jnp.int32)
    src_slabs = jnp.concatenate([src, pad]).reshape(_NW, _CH, _K)
    dst_slabs = jnp.concatenate([dst, pad]).reshape(_NW, _CH, _K)

    x_pad = jnp.zeros((_NPAD, _H), f32).at[:_N].set(x)
    zeros128 = jnp.zeros((_K, _H), f32)
    ones128 = jnp.ones((_K, _H), f32)

    b0r = b0.reshape(1, _H)
    b1r = b1.reshape(1, _H)
    W2p = jnp.zeros((_H, _H), f32).at[:, :3].set(W2)
    b2r = jnp.zeros((1, _H), f32).at[0, :3].set(b2)
    wp_row = jnp.zeros((1, 16), f32).at[0, :3].set(Wp[:, 0])
    wo_row = jnp.zeros((1, 16), f32).at[0, :3].set(Wo[:, 0])
    bpr = bp.reshape(1, 1)
    bor = bo.reshape(1, 1)

    src_flat = src_slabs.reshape(_NW * _CH, _K)
    dst_flat = dst_slabs.reshape(_NW * _CH, _K)

    hist = _sc_degree(src_slabs, dst_slabs, ones128, zeros128)

    z0 = _tc_z0(hist, x_pad, W0)
    acc1 = _sc_spmm(z0, src_flat, dst_flat, zeros128)
    z1 = _tc_mid(hist, acc1, b0r, W1)
    acc2 = _sc_spmm(z1, src_flat, dst_flat, zeros128)
    z2 = _tc_mid(hist, acc2, b1r, W2p)
    acc3 = _sc_spmm(z2, src_flat, dst_flat, zeros128)
    colors_pad = _tc_colors(hist, acc3, b2r)
    sat = _tc_pool(colors_pad, wp_row, bpr, wo_row, bor)

    return colors_pad[:_N, :3], sat[0, 0]


# speedup vs baseline: 1.2705x; 1.2705x over previous
import functools

import jax
import jax.numpy as jnp
from jax import lax
from jax.experimental import pallas as pl
from jax.experimental.pallas import tpu as pltpu
from jax.experimental.pallas import tpu_sc as plsc

_N = 10000
_E = 320000
_H = 128
_NC = 2
_NS = 16
_NW = _NC * _NS
_K = 128
_NPAD = 10240
_ROWS = _NPAD // _NS
_CH = 80
_EPAD = _CH * _NW * _K


def _sc_mesh():
    return plsc.VectorSubcoreMesh(core_axis_name="c", subcore_axis_name="s")


def _sc_degree_body(srcs, dsts, ones_h, zeros_h, out,
                    src_slab, dst_slab, ones_v, acc, sem):
    c = lax.axis_index("c")
    s = lax.axis_index("s")
    wid = s * _NC + c
    r0 = s * _ROWS

    def zstep(i, carry):
        pltpu.sync_copy(zeros_h, acc.at[pl.ds(r0 + i * _K, _K)])
        return carry

    lax.fori_loop(0, _ROWS // _K, zstep, 0)
    pltpu.sync_copy(ones_h, ones_v)
    pltpu.sync_copy(srcs.at[wid], src_slab)
    pltpu.sync_copy(dsts.at[wid], dst_slab)
    plsc.subcore_barrier()

    def step_s(j, carry):
        pltpu.sync_copy(ones_v, acc.at[src_slab.at[j]], add=True)
        return carry

    lax.fori_loop(0, _CH, step_s, 0)
    plsc.subcore_barrier()
    pltpu.sync_copy(acc.at[pl.ds(r0, _ROWS)], out.at[c, 0, pl.ds(r0, _ROWS)])
    lax.fori_loop(0, _ROWS // _K, zstep, 0)
    plsc.subcore_barrier()

    def step_d(j, carry):
        pltpu.sync_copy(ones_v, acc.at[dst_slab.at[j]], add=True)
        return carry

    lax.fori_loop(0, _CH, step_d, 0)
    plsc.subcore_barrier()
    pltpu.sync_copy(acc.at[pl.ds(r0, _ROWS)], out.at[c, 1, pl.ds(r0, _ROWS)])


def _sc_degree(src_slabs, dst_slabs, ones_h, zeros_h):
    kfn = pl.kernel(
        _sc_degree_body,
        mesh=_sc_mesh(),
        out_type=jax.ShapeDtypeStruct((_NC, 2, _NPAD, _H), jnp.float32),
        scratch_types=[
            pltpu.VMEM((_CH, _K), jnp.int32),
            pltpu.VMEM((_CH, _K), jnp.int32),
            pltpu.VMEM((_K, _H), jnp.float32),
            pltpu.VMEM_SHARED((_NPAD, _H), jnp.float32),
            pltpu.SemaphoreType.DMA,
        ],
    )
    return kfn(src_slabs, dst_slabs, ones_h, zeros_h)


_CH0 = 80
_CH1 = 80
_BLK = 40


def _sc_spmm_body(z, srcs, dsts, zeros_h, out,
                  src_slab, dst_slab, rows0, acc, gsem0):
    c = lax.axis_index("c")
    s = lax.axis_index("s")
    r0 = s * _ROWS
    cnt = _CH0 + c * (_CH1 - _CH0)
    start = c * (_NS * _CH0) + s * cnt

    def zstep(i, carry):
        pltpu.sync_copy(zeros_h, acc.at[pl.ds(r0 + i * _K, _K)])
        return carry

    lax.fori_loop(0, _ROWS // _K, zstep, 0)
    plsc.subcore_barrier()

    def blk(b, carry):
        base = start + b * _BLK
        pltpu.sync_copy(srcs.at[pl.ds(base, _BLK)], src_slab)
        pltpu.sync_copy(dsts.at[pl.ds(base, _BLK)], dst_slab)

        def step(j, carry2):
            pltpu.async_copy(z.at[src_slab.at[j]], rows0, gsem0).wait()
            pltpu.sync_copy(rows0, acc.at[dst_slab.at[j]], add=True)
            return carry2

        lax.fori_loop(0, _BLK, step, 0)
        return carry

    lax.fori_loop(0, cnt // _BLK, blk, 0)
    plsc.subcore_barrier()
    pltpu.sync_copy(acc.at[pl.ds(r0, _ROWS)], out.at[c, pl.ds(r0, _ROWS)])


def _sc_spmm(z, src_flat, dst_flat, zeros_h):
    w = z.shape[1]
    kfn = pl.kernel(
        _sc_spmm_body,
        mesh=_sc_mesh(),
        out_type=jax.ShapeDtypeStruct((_NC, _NPAD, w), jnp.float32),
        scratch_types=[
            pltpu.VMEM((_BLK, _K), jnp.int32),
            pltpu.VMEM((_BLK, _K), jnp.int32),
            pltpu.VMEM((_K, w), jnp.float32),
            pltpu.VMEM_SHARED((_NPAD, w), jnp.float32),
            pltpu.SemaphoreType.DMA,
        ],
    )
    return kfn(z, src_flat, dst_flat, zeros_h)


_M = 512


def _norms_from_hist(h4):
    deg_s = h4[0, 0, :, 0:1] + h4[1, 0, :, 0:1]
    deg_d = h4[0, 1, :, 0:1] + h4[1, 1, :, 0:1]
    norm_s = lax.rsqrt(jnp.maximum(deg_s, 1.0))
    norm_d = lax.rsqrt(jnp.maximum(deg_d, 1.0))
    return norm_s, norm_d


def _tc_z0_body(hist_ref, x_ref, w_ref, o_ref):
    norm_s, _ = _norms_from_hist(hist_ref[...])
    o_ref[...] = jnp.dot(x_ref[...] * norm_s, w_ref[...],
                         preferred_element_type=jnp.float32)


def _tc_z0(hist, x, w):
    return pl.pallas_call(
        _tc_z0_body,
        grid=(_NPAD // _M,),
        in_specs=[
            pl.BlockSpec((_NC, 2, _M, _H), lambda i: (0, 0, i, 0)),
            pl.BlockSpec((_M, _H), lambda i: (i, 0)),
            pl.BlockSpec((_H, _H), lambda i: (0, 0)),
        ],
        out_specs=pl.BlockSpec((_M, _H), lambda i: (i, 0)),
        out_shape=jax.ShapeDtypeStruct((_NPAD, _H), jnp.float32),
    )(hist, x, w)


def _tc_mid_body(hist_ref, acc_ref, b_ref, w_ref, o_ref):
    norm_s, norm_d = _norms_from_hist(hist_ref[...])
    a = acc_ref[...]
    y = (a[0] + a[1]) * norm_d + b_ref[...]
    h = jnp.maximum(y, 0.0)
    o_ref[...] = jnp.dot(h * norm_s, w_ref[...],
                         preferred_element_type=jnp.float32)


def _tc_mid(hist, acc, b, w):
    wo = w.shape[1]
    return pl.pallas_call(
        _tc_mid_body,
        grid=(_NPAD // _M,),
        in_specs=[
            pl.BlockSpec((_NC, 2, _M, _H), lambda i: (0, 0, i, 0)),
            pl.BlockSpec((_NC, _M, _H), lambda i: (0, i, 0)),
            pl.BlockSpec((1, _H), lambda i: (0, 0)),
            pl.BlockSpec((_H, wo), lambda i: (0, 0)),
        ],
        out_specs=pl.BlockSpec((_M, wo), lambda i: (i, 0)),
        out_shape=jax.ShapeDtypeStruct((_NPAD, wo), jnp.float32),
    )(hist, acc, b, w)


def _tc_colors_body(hist_ref, acc_ref, b2_ref, colors_ref):
    h4 = hist_ref[...]
    deg_d = h4[0, 1, :, 0:1] + h4[1, 1, :, 0:1]
    norm_d = lax.rsqrt(jnp.maximum(deg_d, 1.0))
    a = acc_ref[...]
    y = (a[0] + a[1]) * norm_d + b2_ref[...]
    neg = jnp.float32(-1e30)
    col = lax.broadcasted_iota(jnp.int32, (_M, _H), 1)
    colmask = col < 3
    logits = jnp.where(colmask, y, neg)
    m = jnp.max(logits, axis=1, keepdims=True)
    e = jnp.where(colmask, jnp.exp(logits - m), 0.0)
    colors = e / jnp.sum(e, axis=1, keepdims=True)
    colors_ref[...] = colors[:, :16]


def _tc_colors(hist, acc, b2):
    return pl.pallas_call(
        _tc_colors_body,
        grid=(_NPAD // _M,),
        in_specs=[
            pl.BlockSpec((_NC, 2, _M, _H), lambda i: (0, 0, i, 0)),
            pl.BlockSpec((_NC, _M, _H), lambda i: (0, i, 0)),
            pl.BlockSpec((1, _H), lambda i: (0, 0)),
        ],
        out_specs=pl.BlockSpec((_M, 16), lambda i: (i, 0)),
        out_shape=jax.ShapeDtypeStruct((_NPAD, 16), jnp.float32),
    )(hist, acc, b2)


def _tc_pool_body(colors_ref, wp_ref, bp_ref, wo_ref, bo_ref, sat_ref):
    colors = colors_ref[...]
    neg = jnp.float32(-1e30)
    gate = jnp.sum(colors * wp_ref[...], axis=1, keepdims=True) + bp_ref[0, 0]
    row = lax.broadcasted_iota(jnp.int32, (_NPAD, 1), 0)
    rowmask = row < _N
    glog = jnp.where(rowmask, gate, neg)
    gm = jnp.max(glog, axis=0, keepdims=True)
    ge = jnp.where(rowmask, jnp.exp(glog - gm), 0.0)
    gw = ge / jnp.sum(ge, axis=0, keepdims=True)
    readout = jnp.sum(gw * colors, axis=0, keepdims=True)
    logit = jnp.sum(readout * wo_ref[...]) + bo_ref[0, 0]
    sat = 1.0 / (1.0 + jnp.exp(-logit))
    sat_ref[...] = jnp.reshape(sat, (1, 1))


def _tc_pool(colors, wp, bp, wo, bo):
    return pl.pallas_call(
        _tc_pool_body,
        out_shape=jax.ShapeDtypeStruct((1, 1), jnp.float32),
    )(colors, wp, bp, wo, bo)


def kernel(x, edge_index, W0, b0, W1, b1, W2, b2, Wp, bp, Wo, bo):
    f32 = jnp.float32
    src = edge_index[0]
    dst = edge_index[1]
    pad = jnp.full((_EPAD - _E,), _N, dtype=jnp.int32)
    src_slabs = jnp.concatenate([src, pad]).reshape(_NW, _CH, _K)
    dst_slabs = jnp.concatenate([dst, pad]).reshape(_NW, _CH, _K)

    x_pad = jnp.zeros((_NPAD, _H), f32).at[:_N].set(x)
    zeros128 = jnp.zeros((_K, _H), f32)
    ones128 = jnp.ones((_K, _H), f32)

    b0r = b0.reshape(1, _H)
    b1r = b1.reshape(1, _H)
    W2p = jnp.zeros((_H, _H), f32).at[:, :3].set(W2)
    b2r = jnp.zeros((1, _H), f32).at[0, :3].set(b2)
    wp_row = jnp.zeros((1, 16), f32).at[0, :3].set(Wp[:, 0])
    wo_row = jnp.zeros((1, 16), f32).at[0, :3].set(Wo[:, 0])
    bpr = bp.reshape(1, 1)
    bor = bo.reshape(1, 1)

    src_flat = src_slabs.reshape(_NW * _CH, _K)
    dst_flat = dst_slabs.reshape(_NW * _CH, _K)

    hist = _sc_degree(src_slabs, dst_slabs, ones128, zeros128)

    z0 = _tc_z0(hist, x_pad, W0)
    acc1 = _sc_spmm(z0, src_flat, dst_flat, zeros128)
    z1 = _tc_mid(hist, acc1, b0r, W1)
    acc2 = _sc_spmm(z1, src_flat, dst_flat, zeros128)
    z2 = _tc_mid(hist, acc2, b1r, W2p)
    acc3 = _sc_spmm(z2, src_flat, dst_flat, zeros128)
    colors_pad = _tc_colors(hist, acc3, b2r)
    sat = _tc_pool(colors_pad, wp_row, bpr, wo_row, bor)

    return colors_pad[:_N, :3], sat[0, 0]


# speedup vs baseline: 1.4276x; 1.1236x over previous
import functools

import jax
import jax.numpy as jnp
from jax import lax
from jax.experimental import pallas as pl
from jax.experimental.pallas import tpu as pltpu
from jax.experimental.pallas import tpu_sc as plsc

_N = 10000
_E = 320000
_H = 128
_NC = 2
_NS = 16
_NW = _NC * _NS
_K = 128
_NPAD = 10240
_ROWS = _NPAD // _NS
_CH = 80
_EPAD = _CH * _NW * _K


def _sc_mesh():
    return plsc.VectorSubcoreMesh(core_axis_name="c", subcore_axis_name="s")


def _sc_degree_body(srcs, dsts, ones_h, zeros_h, out,
                    src_slab, dst_slab, ones_v, acc, sem):
    c = lax.axis_index("c")
    s = lax.axis_index("s")
    wid = s * _NC + c
    r0 = s * _ROWS

    pltpu.sync_copy(zeros_h, acc.at[pl.ds(r0, _ROWS)])
    pltpu.sync_copy(ones_h, ones_v)
    pltpu.sync_copy(srcs.at[wid], src_slab)
    pltpu.sync_copy(dsts.at[wid], dst_slab)
    plsc.subcore_barrier()

    def step_s(j, carry):
        pltpu.sync_copy(ones_v, acc.at[src_slab.at[j]], add=True)
        return carry

    lax.fori_loop(0, _CH, step_s, 0)
    plsc.subcore_barrier()
    pltpu.sync_copy(acc.at[pl.ds(r0, _ROWS)], out.at[c, 0, pl.ds(r0, _ROWS)])
    pltpu.sync_copy(zeros_h, acc.at[pl.ds(r0, _ROWS)])
    plsc.subcore_barrier()

    def step_d(j, carry):
        pltpu.sync_copy(ones_v, acc.at[dst_slab.at[j]], add=True)
        return carry

    lax.fori_loop(0, _CH, step_d, 0)
    plsc.subcore_barrier()
    pltpu.sync_copy(acc.at[pl.ds(r0, _ROWS)], out.at[c, 1, pl.ds(r0, _ROWS)])


def _sc_degree(src_slabs, dst_slabs, ones_h, zeros_h):
    kfn = pl.kernel(
        _sc_degree_body,
        mesh=_sc_mesh(),
        out_type=jax.ShapeDtypeStruct((_NC, 2, _NPAD, _H), jnp.float32),
        scratch_types=[
            pltpu.VMEM((_CH, _K), jnp.int32),
            pltpu.VMEM((_CH, _K), jnp.int32),
            pltpu.VMEM((_K, _H), jnp.float32),
            pltpu.VMEM_SHARED((_NPAD, _H), jnp.float32),
            pltpu.SemaphoreType.DMA,
        ],
    )
    return kfn(src_slabs, dst_slabs, ones_h, zeros_h)


_CH0 = 80
_CH1 = 80
_BLK = 40


def _sc_spmm_body(z, srcs, dsts, zeros_h, out,
                  src_slab, dst_slab, rows0, acc, gsem0):
    c = lax.axis_index("c")
    s = lax.axis_index("s")
    r0 = s * _ROWS
    cnt = _CH0 + c * (_CH1 - _CH0)
    start = c * (_NS * _CH0) + s * cnt

    pltpu.sync_copy(zeros_h, acc.at[pl.ds(r0, _ROWS)])
    plsc.subcore_barrier()

    def blk(b, carry):
        base = start + b * _BLK
        pltpu.sync_copy(srcs.at[pl.ds(base, _BLK)], src_slab)
        pltpu.sync_copy(dsts.at[pl.ds(base, _BLK)], dst_slab)

        def step(j, carry2):
            pltpu.async_copy(z.at[src_slab.at[j]], rows0, gsem0).wait()
            pltpu.sync_copy(rows0, acc.at[dst_slab.at[j]], add=True)
            return carry2

        lax.fori_loop(0, _BLK, step, 0)
        return carry

    lax.fori_loop(0, cnt // _BLK, blk, 0)
    plsc.subcore_barrier()
    pltpu.sync_copy(acc.at[pl.ds(r0, _ROWS)], out.at[c, pl.ds(r0, _ROWS)])


def _sc_spmm(z, src_flat, dst_flat, zeros_h):
    w = z.shape[1]
    kfn = pl.kernel(
        _sc_spmm_body,
        mesh=_sc_mesh(),
        out_type=jax.ShapeDtypeStruct((_NC, _NPAD, w), jnp.float32),
        scratch_types=[
            pltpu.VMEM((_BLK, _K), jnp.int32),
            pltpu.VMEM((_BLK, _K), jnp.int32),
            pltpu.VMEM((_K, w), jnp.float32),
            pltpu.VMEM_SHARED((_NPAD, w), jnp.float32),
            pltpu.SemaphoreType.DMA,
        ],
    )
    return kfn(z, src_flat, dst_flat, zeros_h)


_M = 512


def _norms_from_hist(h4):
    deg_s = h4[0, 0, :, 0:1] + h4[1, 0, :, 0:1]
    deg_d = h4[0, 1, :, 0:1] + h4[1, 1, :, 0:1]
    norm_s = lax.rsqrt(jnp.maximum(deg_s, 1.0))
    norm_d = lax.rsqrt(jnp.maximum(deg_d, 1.0))
    return norm_s, norm_d


def _tc_z0_body(hist_ref, x_ref, w_ref, o_ref):
    norm_s, _ = _norms_from_hist(hist_ref[...])
    o_ref[...] = jnp.dot(x_ref[...] * norm_s, w_ref[...],
                         preferred_element_type=jnp.float32)


def _tc_z0(hist, x, w):
    return pl.pallas_call(
        _tc_z0_body,
        grid=(_NPAD // _M,),
        in_specs=[
            pl.BlockSpec((_NC, 2, _M, _H), lambda i: (0, 0, i, 0)),
            pl.BlockSpec((_M, _H), lambda i: (i, 0)),
            pl.BlockSpec((_H, _H), lambda i: (0, 0)),
        ],
        out_specs=pl.BlockSpec((_M, _H), lambda i: (i, 0)),
        out_shape=jax.ShapeDtypeStruct((_NPAD, _H), jnp.float32),
    )(hist, x, w)


def _tc_mid_body(hist_ref, acc_ref, b_ref, w_ref, o_ref):
    norm_s, norm_d = _norms_from_hist(hist_ref[...])
    a = acc_ref[...]
    y = (a[0] + a[1]) * norm_d + b_ref[...]
    h = jnp.maximum(y, 0.0)
    o_ref[...] = jnp.dot(h * norm_s, w_ref[...],
                         preferred_element_type=jnp.float32)


def _tc_mid(hist, acc, b, w):
    wo = w.shape[1]
    return pl.pallas_call(
        _tc_mid_body,
        grid=(_NPAD // _M,),
        in_specs=[
            pl.BlockSpec((_NC, 2, _M, _H), lambda i: (0, 0, i, 0)),
            pl.BlockSpec((_NC, _M, _H), lambda i: (0, i, 0)),
            pl.BlockSpec((1, _H), lambda i: (0, 0)),
            pl.BlockSpec((_H, wo), lambda i: (0, 0)),
        ],
        out_specs=pl.BlockSpec((_M, wo), lambda i: (i, 0)),
        out_shape=jax.ShapeDtypeStruct((_NPAD, wo), jnp.float32),
    )(hist, acc, b, w)


def _tc_colors_body(hist_ref, acc_ref, b2_ref, colors_ref):
    h4 = hist_ref[...]
    deg_d = h4[0, 1, :, 0:1] + h4[1, 1, :, 0:1]
    norm_d = lax.rsqrt(jnp.maximum(deg_d, 1.0))
    a = acc_ref[...]
    y = (a[0] + a[1]) * norm_d + b2_ref[...]
    neg = jnp.float32(-1e30)
    col = lax.broadcasted_iota(jnp.int32, (_M, _H), 1)
    colmask = col < 3
    logits = jnp.where(colmask, y, neg)
    m = jnp.max(logits, axis=1, keepdims=True)
    e = jnp.where(colmask, jnp.exp(logits - m), 0.0)
    colors = e / jnp.sum(e, axis=1, keepdims=True)
    colors_ref[...] = colors[:, :16]


def _tc_colors(hist, acc, b2):
    return pl.pallas_call(
        _tc_colors_body,
        grid=(_NPAD // _M,),
        in_specs=[
            pl.BlockSpec((_NC, 2, _M, _H), lambda i: (0, 0, i, 0)),
            pl.BlockSpec((_NC, _M, _H), lambda i: (0, i, 0)),
            pl.BlockSpec((1, _H), lambda i: (0, 0)),
        ],
        out_specs=pl.BlockSpec((_M, 16), lambda i: (i, 0)),
        out_shape=jax.ShapeDtypeStruct((_NPAD, 16), jnp.float32),
    )(hist, acc, b2)


def _tc_pool_body(colors_ref, wp_ref, bp_ref, wo_ref, bo_ref, sat_ref):
    colors = colors_ref[...]
    neg = jnp.float32(-1e30)
    gate = jnp.sum(colors * wp_ref[...], axis=1, keepdims=True) + bp_ref[0, 0]
    row = lax.broadcasted_iota(jnp.int32, (_NPAD, 1), 0)
    rowmask = row < _N
    glog = jnp.where(rowmask, gate, neg)
    gm = jnp.max(glog, axis=0, keepdims=True)
    ge = jnp.where(rowmask, jnp.exp(glog - gm), 0.0)
    gw = ge / jnp.sum(ge, axis=0, keepdims=True)
    readout = jnp.sum(gw * colors, axis=0, keepdims=True)
    logit = jnp.sum(readout * wo_ref[...]) + bo_ref[0, 0]
    sat = 1.0 / (1.0 + jnp.exp(-logit))
    sat_ref[...] = jnp.reshape(sat, (1, 1))


def _tc_pool(colors, wp, bp, wo, bo):
    return pl.pallas_call(
        _tc_pool_body,
        out_shape=jax.ShapeDtypeStruct((1, 1), jnp.float32),
    )(colors, wp, bp, wo, bo)


def kernel(x, edge_index, W0, b0, W1, b1, W2, b2, Wp, bp, Wo, bo):
    f32 = jnp.float32
    src = edge_index[0]
    dst = edge_index[1]
    pad = jnp.full((_EPAD - _E,), _N, dtype=jnp.int32)
    src_slabs = jnp.concatenate([src, pad]).reshape(_NW, _CH, _K)
    dst_slabs = jnp.concatenate([dst, pad]).reshape(_NW, _CH, _K)

    x_pad = jnp.zeros((_NPAD, _H), f32).at[:_N].set(x)
    zeros128 = jnp.zeros((_ROWS, _H), f32)
    ones128 = jnp.ones((_K, _H), f32)

    b0r = b0.reshape(1, _H)
    b1r = b1.reshape(1, _H)
    W2p = jnp.zeros((_H, _H), f32).at[:, :3].set(W2)
    b2r = jnp.zeros((1, _H), f32).at[0, :3].set(b2)
    wp_row = jnp.zeros((1, 16), f32).at[0, :3].set(Wp[:, 0])
    wo_row = jnp.zeros((1, 16), f32).at[0, :3].set(Wo[:, 0])
    bpr = bp.reshape(1, 1)
    bor = bo.reshape(1, 1)

    src_flat = src_slabs.reshape(_NW * _CH, _K)
    dst_flat = dst_slabs.reshape(_NW * _CH, _K)

    hist = _sc_degree(src_slabs, dst_slabs, ones128, zeros128)

    z0 = _tc_z0(hist, x_pad, W0)
    acc1 = _sc_spmm(z0, src_flat, dst_flat, zeros128)
    z1 = _tc_mid(hist, acc1, b0r, W1)
    acc2 = _sc_spmm(z1, src_flat, dst_flat, zeros128)
    z2 = _tc_mid(hist, acc2, b1r, W2p)
    acc3 = _sc_spmm(z2, src_flat, dst_flat, zeros128)
    colors_pad = _tc_colors(hist, acc3, b2r)
    sat = _tc_pool(colors_pad, wp_row, bpr, wo_row, bor)

    return colors_pad[:_N, :3], sat[0, 0]


# speedup vs baseline: 3.2012x; 2.2424x over previous
import functools

import jax
import jax.numpy as jnp
from jax import lax
from jax.experimental import pallas as pl
from jax.experimental.pallas import tpu as pltpu
from jax.experimental.pallas import tpu_sc as plsc

_N = 10000
_E = 320000
_H = 128
_NC = 2
_NS = 16
_NW = _NC * _NS
_K = 128
_NPAD = 10240
_ROWS = _NPAD // _NS
_CH = 80
_EPAD = _CH * _NW * _K


def _sc_mesh():
    return plsc.VectorSubcoreMesh(core_axis_name="c", subcore_axis_name="s")


def _sc_degree_body(srcs, dsts, ones_h, zeros_h, out,
                    src_slab, dst_slab, ones_v, acc, sem):
    c = lax.axis_index("c")
    s = lax.axis_index("s")
    wid = s * _NC + c
    r0 = s * _ROWS

    pltpu.sync_copy(zeros_h, acc.at[pl.ds(r0, _ROWS)])
    pltpu.sync_copy(ones_h, ones_v)
    pltpu.sync_copy(srcs.at[wid], src_slab)
    pltpu.sync_copy(dsts.at[wid], dst_slab)
    plsc.subcore_barrier()

    def step_s(j, carry):
        pltpu.sync_copy(ones_v, acc.at[src_slab.at[j]], add=True)
        return carry

    lax.fori_loop(0, _CH, step_s, 0)
    plsc.subcore_barrier()
    pltpu.sync_copy(acc.at[pl.ds(r0, _ROWS)], out.at[c, 0, pl.ds(r0, _ROWS)])
    pltpu.sync_copy(zeros_h, acc.at[pl.ds(r0, _ROWS)])
    plsc.subcore_barrier()

    def step_d(j, carry):
        pltpu.sync_copy(ones_v, acc.at[dst_slab.at[j]], add=True)
        return carry

    lax.fori_loop(0, _CH, step_d, 0)
    plsc.subcore_barrier()
    pltpu.sync_copy(acc.at[pl.ds(r0, _ROWS)], out.at[c, 1, pl.ds(r0, _ROWS)])


def _sc_degree(src_slabs, dst_slabs, ones_h, zeros_h):
    kfn = pl.kernel(
        _sc_degree_body,
        mesh=_sc_mesh(),
        out_type=jax.ShapeDtypeStruct((_NC, 2, _NPAD, _H), jnp.float32),
        scratch_types=[
            pltpu.VMEM((_CH, _K), jnp.int32),
            pltpu.VMEM((_CH, _K), jnp.int32),
            pltpu.VMEM((_K, _H), jnp.float32),
            pltpu.VMEM_SHARED((_NPAD, _H), jnp.float32),
            pltpu.SemaphoreType.DMA,
        ],
    )
    return kfn(src_slabs, dst_slabs, ones_h, zeros_h)


_CH0 = 80
_CH1 = 80
_BLK = 40


def _sc_spmm_body(z, srcs, dsts, zeros_h, out,
                  src_slab, dst_slab, rows0, acc, gsem0):
    c = lax.axis_index("c")
    s = lax.axis_index("s")
    r0 = s * _ROWS
    cnt = _CH0 + c * (_CH1 - _CH0)
    start = c * (_NS * _CH0) + s * cnt

    pltpu.sync_copy(zeros_h, acc.at[pl.ds(r0, _ROWS)])
    plsc.subcore_barrier()

    def blk(b, carry):
        base = start + b * _BLK
        pltpu.sync_copy(srcs.at[pl.ds(base, _BLK)], src_slab)
        pltpu.sync_copy(dsts.at[pl.ds(base, _BLK)], dst_slab)

        def step(j, carry2):
            pltpu.async_copy(z.at[src_slab.at[j]], rows0, gsem0).wait()
            pltpu.sync_copy(rows0, acc.at[dst_slab.at[j]], add=True)
            return carry2

        lax.fori_loop(0, _BLK, step, 0)
        return carry

    lax.fori_loop(0, cnt // _BLK, blk, 0)
    plsc.subcore_barrier()
    pltpu.sync_copy(acc.at[pl.ds(r0, _ROWS)], out.at[c, pl.ds(r0, _ROWS)])


def _sc_spmm(z, src_flat, dst_flat, zeros_h):
    w = z.shape[1]
    kfn = pl.kernel(
        _sc_spmm_body,
        mesh=_sc_mesh(),
        out_type=jax.ShapeDtypeStruct((_NC, _NPAD, w), jnp.float32),
        scratch_types=[
            pltpu.VMEM((_BLK, _K), jnp.int32),
            pltpu.VMEM((_BLK, _K), jnp.int32),
            pltpu.VMEM((_K, w), jnp.float32),
            pltpu.VMEM_SHARED((_NPAD, w), jnp.float32),
            pltpu.SemaphoreType.DMA,
        ],
    )
    return kfn(z, src_flat, dst_flat, zeros_h)


_M = 512


def _norms_from_hist(h4):
    deg_s = h4[0, 0, :, 0:1] + h4[1, 0, :, 0:1]
    deg_d = h4[0, 1, :, 0:1] + h4[1, 1, :, 0:1]
    norm_s = lax.rsqrt(jnp.maximum(deg_s, 1.0))
    norm_d = lax.rsqrt(jnp.maximum(deg_d, 1.0))
    return norm_s, norm_d


def _tc_z0_body(hist_ref, x_ref, w_ref, o_ref):
    norm_s, _ = _norms_from_hist(hist_ref[...])
    o_ref[...] = jnp.dot(x_ref[...] * norm_s, w_ref[...],
                         preferred_element_type=jnp.float32)


def _tc_z0(hist, x, w):
    return pl.pallas_call(
        _tc_z0_body,
        grid=(_NPAD // _M,),
        in_specs=[
            pl.BlockSpec((_NC, 2, _M, _H), lambda i: (0, 0, i, 0)),
            pl.BlockSpec((_M, _H), lambda i: (i, 0)),
            pl.BlockSpec((_H, _H), lambda i: (0, 0)),
        ],
        out_specs=pl.BlockSpec((_M, _H), lambda i: (i, 0)),
        out_shape=jax.ShapeDtypeStruct((_NPAD, _H), jnp.float32),
    )(hist, x, w)


def _tc_mid_body(hist_ref, acc_ref, b_ref, w_ref, o_ref):
    norm_s, norm_d = _norms_from_hist(hist_ref[...])
    a = acc_ref[...]
    y = (a[0] + a[1]) * norm_d + b_ref[...]
    h = jnp.maximum(y, 0.0)
    o_ref[...] = jnp.dot(h * norm_s, w_ref[...],
                         preferred_element_type=jnp.float32)


def _tc_mid(hist, acc, b, w):
    wo = w.shape[1]
    return pl.pallas_call(
        _tc_mid_body,
        grid=(_NPAD // _M,),
        in_specs=[
            pl.BlockSpec((_NC, 2, _M, _H), lambda i: (0, 0, i, 0)),
            pl.BlockSpec((_NC, _M, _H), lambda i: (0, i, 0)),
            pl.BlockSpec((1, _H), lambda i: (0, 0)),
            pl.BlockSpec((_H, wo), lambda i: (0, 0)),
        ],
        out_specs=pl.BlockSpec((_M, wo), lambda i: (i, 0)),
        out_shape=jax.ShapeDtypeStruct((_NPAD, wo), jnp.float32),
    )(hist, acc, b, w)


def _tc_colors_body(hist_ref, acc_ref, b2_ref, colors_ref):
    h4 = hist_ref[...]
    deg_d = h4[0, 1, :, 0:1] + h4[1, 1, :, 0:1]
    norm_d = lax.rsqrt(jnp.maximum(deg_d, 1.0))
    a = acc_ref[...]
    y = (a[0] + a[1]) * norm_d + b2_ref[...]
    neg = jnp.float32(-1e30)
    col = lax.broadcasted_iota(jnp.int32, (_M, _H), 1)
    colmask = col < 3
    logits = jnp.where(colmask, y, neg)
    m = jnp.max(logits, axis=1, keepdims=True)
    e = jnp.where(colmask, jnp.exp(logits - m), 0.0)
    colors = e / jnp.sum(e, axis=1, keepdims=True)
    colors_ref[...] = colors[:, :16]


def _tc_colors(hist, acc, b2):
    return pl.pallas_call(
        _tc_colors_body,
        grid=(_NPAD // _M,),
        in_specs=[
            pl.BlockSpec((_NC, 2, _M, _H), lambda i: (0, 0, i, 0)),
            pl.BlockSpec((_NC, _M, _H), lambda i: (0, i, 0)),
            pl.BlockSpec((1, _H), lambda i: (0, 0)),
        ],
        out_specs=pl.BlockSpec((_M, 16), lambda i: (i, 0)),
        out_shape=jax.ShapeDtypeStruct((_NPAD, 16), jnp.float32),
    )(hist, acc, b2)


def _tc_pool_body(colors_ref, wp_ref, bp_ref, wo_ref, bo_ref, sat_ref):
    colors = colors_ref[...]
    neg = jnp.float32(-1e30)
    gate = jnp.sum(colors * wp_ref[...], axis=1, keepdims=True) + bp_ref[0, 0]
    row = lax.broadcasted_iota(jnp.int32, (_NPAD, 1), 0)
    rowmask = row < _N
    glog = jnp.where(rowmask, gate, neg)
    gm = jnp.max(glog, axis=0, keepdims=True)
    ge = jnp.where(rowmask, jnp.exp(glog - gm), 0.0)
    gw = ge / jnp.sum(ge, axis=0, keepdims=True)
    readout = jnp.sum(gw * colors, axis=0, keepdims=True)
    logit = jnp.sum(readout * wo_ref[...]) + bo_ref[0, 0]
    sat = 1.0 / (1.0 + jnp.exp(-logit))
    sat_ref[...] = jnp.reshape(sat, (1, 1))


def _tc_pool(colors, wp, bp, wo, bo):
    return pl.pallas_call(
        _tc_pool_body,
        out_shape=jax.ShapeDtypeStruct((1, 1), jnp.float32),
    )(colors, wp, bp, wo, bo)


def kernel(x, edge_index, W0, b0, W1, b1, W2, b2, Wp, bp, Wo, bo):
    f32 = jnp.float32
    src = edge_index[0]
    dst = edge_index[1]
    pad = _N + (jnp.arange(_EPAD - _E, dtype=jnp.int32) % (_NPAD - _N))
    src_slabs = jnp.concatenate([src, pad]).reshape(_NW, _CH, _K)
    dst_slabs = jnp.concatenate([dst, pad]).reshape(_NW, _CH, _K)

    x_pad = jnp.zeros((_NPAD, _H), f32).at[:_N].set(x)
    zeros128 = jnp.zeros((_ROWS, _H), f32)
    ones128 = jnp.ones((_K, _H), f32)

    b0r = b0.reshape(1, _H)
    b1r = b1.reshape(1, _H)
    W2p = jnp.zeros((_H, _H), f32).at[:, :3].set(W2)
    b2r = jnp.zeros((1, _H), f32).at[0, :3].set(b2)
    wp_row = jnp.zeros((1, 16), f32).at[0, :3].set(Wp[:, 0])
    wo_row = jnp.zeros((1, 16), f32).at[0, :3].set(Wo[:, 0])
    bpr = bp.reshape(1, 1)
    bor = bo.reshape(1, 1)

    src_flat = src_slabs.reshape(_NW * _CH, _K)
    dst_flat = dst_slabs.reshape(_NW * _CH, _K)

    hist = _sc_degree(src_slabs, dst_slabs, ones128, zeros128)

    z0 = _tc_z0(hist, x_pad, W0)
    acc1 = _sc_spmm(z0, src_flat, dst_flat, zeros128)
    z1 = _tc_mid(hist, acc1, b0r, W1)
    acc2 = _sc_spmm(z1, src_flat, dst_flat, zeros128)
    z2 = _tc_mid(hist, acc2, b1r, W2p)
    acc3 = _sc_spmm(z2, src_flat, dst_flat, zeros128)
    colors_pad = _tc_colors(hist, acc3, b2r)
    sat = _tc_pool(colors_pad, wp_row, bpr, wo_row, bor)

    return colors_pad[:_N, :3], sat[0, 0]


# speedup vs baseline: 4.3443x; 1.3571x over previous
import functools

import jax
import jax.numpy as jnp
from jax import lax
from jax.experimental import pallas as pl
from jax.experimental.pallas import tpu as pltpu
from jax.experimental.pallas import tpu_sc as plsc

_N = 10000
_E = 320000
_H = 128
_NC = 2
_NS = 16
_NW = _NC * _NS
_K = 128
_NPAD = 10240
_ROWS = _NPAD // _NS
_CH = 80
_EPAD = _CH * _NW * _K


def _sc_mesh():
    return plsc.VectorSubcoreMesh(core_axis_name="c", subcore_axis_name="s")


def _sc_degree_body(srcs, dsts, ones_h, zeros_h, out,
                    src_slab, dst_slab, ones_v, acc, sem):
    c = lax.axis_index("c")
    s = lax.axis_index("s")
    wid = s * _NC + c
    r0 = s * _ROWS

    pltpu.sync_copy(zeros_h, acc.at[pl.ds(r0, _ROWS)])
    pltpu.sync_copy(ones_h, ones_v)
    pltpu.sync_copy(srcs.at[wid], src_slab)
    pltpu.sync_copy(dsts.at[wid], dst_slab)
    plsc.subcore_barrier()

    def step_s(j, carry):
        pltpu.sync_copy(ones_v, acc.at[src_slab.at[j]], add=True)
        return carry

    lax.fori_loop(0, _CH, step_s, 0)
    plsc.subcore_barrier()
    pltpu.sync_copy(acc.at[pl.ds(r0, _ROWS)], out.at[c, 0, pl.ds(r0, _ROWS)])
    pltpu.sync_copy(zeros_h, acc.at[pl.ds(r0, _ROWS)])
    plsc.subcore_barrier()

    def step_d(j, carry):
        pltpu.sync_copy(ones_v, acc.at[dst_slab.at[j]], add=True)
        return carry

    lax.fori_loop(0, _CH, step_d, 0)
    plsc.subcore_barrier()
    pltpu.sync_copy(acc.at[pl.ds(r0, _ROWS)], out.at[c, 1, pl.ds(r0, _ROWS)])


def _sc_degree(src_slabs, dst_slabs, ones_h, zeros_h):
    kfn = pl.kernel(
        _sc_degree_body,
        mesh=_sc_mesh(),
        out_type=jax.ShapeDtypeStruct((_NC, 2, _NPAD, 16), jnp.float32),
        scratch_types=[
            pltpu.VMEM((_CH, _K), jnp.int32),
            pltpu.VMEM((_CH, _K), jnp.int32),
            pltpu.VMEM((_K, 16), jnp.float32),
            pltpu.VMEM_SHARED((_NPAD, 16), jnp.float32),
            pltpu.SemaphoreType.DMA,
        ],
        compiler_params=pltpu.CompilerParams(use_tc_tiling_on_sc=False),
    )
    return kfn(src_slabs, dst_slabs, ones_h, zeros_h)


_CH0 = 80
_CH1 = 80
_BLK = 16


def _sc_spmm_body(z, srcs, dsts, zeros_h, out,
                  src_slab, dst_slab, rows0, rows1, acc, gsem0, gsem1):
    c = lax.axis_index("c")
    s = lax.axis_index("s")
    r0 = s * _ROWS
    cnt = _CH0 + c * (_CH1 - _CH0)
    start = c * (_NS * _CH0) + s * cnt

    pltpu.sync_copy(zeros_h, acc.at[pl.ds(r0, _ROWS)])
    plsc.subcore_barrier()

    def blk(b, carry):
        base = start + b * _BLK
        pltpu.sync_copy(srcs.at[pl.ds(base, _BLK)], src_slab)
        pltpu.sync_copy(dsts.at[pl.ds(base, _BLK)], dst_slab)
        pltpu.async_copy(z.at[src_slab.at[0]], rows0, gsem0)

        def step(jj, carry2):
            j0 = 2 * jj
            j1 = j0 + 1
            pltpu.make_async_copy(z.at[src_slab.at[j0]], rows0, gsem0).wait()
            pltpu.async_copy(z.at[src_slab.at[j1]], rows1, gsem1)
            pltpu.sync_copy(rows0, acc.at[dst_slab.at[j0]], add=True)
            pltpu.make_async_copy(z.at[src_slab.at[j1]], rows1, gsem1).wait()

            @pl.when(j1 + 1 < _BLK)
            def _():
                pltpu.async_copy(z.at[src_slab.at[j1 + 1]], rows0, gsem0)

            pltpu.sync_copy(rows1, acc.at[dst_slab.at[j1]], add=True)
            return carry2

        lax.fori_loop(0, _BLK // 2, step, 0)
        return carry

    lax.fori_loop(0, cnt // _BLK, blk, 0)
    plsc.subcore_barrier()
    pltpu.sync_copy(acc.at[pl.ds(r0, _ROWS)], out.at[c, pl.ds(r0, _ROWS)])


def _sc_spmm(z, src_flat, dst_flat, zeros_h):
    w = z.shape[1]
    kfn = pl.kernel(
        _sc_spmm_body,
        mesh=_sc_mesh(),
        out_type=jax.ShapeDtypeStruct((_NC, _NPAD, w), jnp.float32),
        scratch_types=[
            pltpu.VMEM((_BLK, _K), jnp.int32),
            pltpu.VMEM((_BLK, _K), jnp.int32),
            pltpu.VMEM((_K, w), jnp.float32),
            pltpu.VMEM((_K, w), jnp.float32),
            pltpu.VMEM_SHARED((_NPAD, w), jnp.float32),
            pltpu.SemaphoreType.DMA,
            pltpu.SemaphoreType.DMA,
        ],
    )
    return kfn(z, src_flat, dst_flat, zeros_h)


_M = 512


def _norms_from_hist(h4):
    deg_s = h4[0, 0, :, 0:1] + h4[1, 0, :, 0:1]
    deg_d = h4[0, 1, :, 0:1] + h4[1, 1, :, 0:1]
    norm_s = lax.rsqrt(jnp.maximum(deg_s, 1.0))
    norm_d = lax.rsqrt(jnp.maximum(deg_d, 1.0))
    return norm_s, norm_d


def _tc_z0_body(hist_ref, x_ref, w_ref, o_ref):
    norm_s, _ = _norms_from_hist(hist_ref[...])
    o_ref[...] = jnp.dot(x_ref[...] * norm_s, w_ref[...],
                         preferred_element_type=jnp.float32)


def _tc_z0(hist, x, w):
    return pl.pallas_call(
        _tc_z0_body,
        grid=(_NPAD // _M,),
        in_specs=[
            pl.BlockSpec((_NC, 2, _M, 16), lambda i: (0, 0, i, 0)),
            pl.BlockSpec((_M, _H), lambda i: (i, 0)),
            pl.BlockSpec((_H, _H), lambda i: (0, 0)),
        ],
        out_specs=pl.BlockSpec((_M, _H), lambda i: (i, 0)),
        out_shape=jax.ShapeDtypeStruct((_NPAD, _H), jnp.float32),
    )(hist, x, w)


def _tc_mid_body(hist_ref, acc_ref, b_ref, w_ref, o_ref):
    norm_s, norm_d = _norms_from_hist(hist_ref[...])
    a = acc_ref[...]
    y = (a[0] + a[1]) * norm_d + b_ref[...]
    h = jnp.maximum(y, 0.0)
    o_ref[...] = jnp.dot(h * norm_s, w_ref[...],
                         preferred_element_type=jnp.float32)


def _tc_mid(hist, acc, b, w):
    wo = w.shape[1]
    return pl.pallas_call(
        _tc_mid_body,
        grid=(_NPAD // _M,),
        in_specs=[
            pl.BlockSpec((_NC, 2, _M, 16), lambda i: (0, 0, i, 0)),
            pl.BlockSpec((_NC, _M, _H), lambda i: (0, i, 0)),
            pl.BlockSpec((1, _H), lambda i: (0, 0)),
            pl.BlockSpec((_H, wo), lambda i: (0, 0)),
        ],
        out_specs=pl.BlockSpec((_M, wo), lambda i: (i, 0)),
        out_shape=jax.ShapeDtypeStruct((_NPAD, wo), jnp.float32),
    )(hist, acc, b, w)


def _tc_colors_body(hist_ref, acc_ref, b2_ref, colors_ref):
    h4 = hist_ref[...]
    deg_d = h4[0, 1, :, 0:1] + h4[1, 1, :, 0:1]
    norm_d = lax.rsqrt(jnp.maximum(deg_d, 1.0))
    a = acc_ref[...]
    y = (a[0] + a[1]) * norm_d + b2_ref[...]
    neg = jnp.float32(-1e30)
    col = lax.broadcasted_iota(jnp.int32, (_M, _H), 1)
    colmask = col < 3
    logits = jnp.where(colmask, y, neg)
    m = jnp.max(logits, axis=1, keepdims=True)
    e = jnp.where(colmask, jnp.exp(logits - m), 0.0)
    colors = e / jnp.sum(e, axis=1, keepdims=True)
    colors_ref[...] = colors[:, :16]


def _tc_colors(hist, acc, b2):
    return pl.pallas_call(
        _tc_colors_body,
        grid=(_NPAD // _M,),
        in_specs=[
            pl.BlockSpec((_NC, 2, _M, 16), lambda i: (0, 0, i, 0)),
            pl.BlockSpec((_NC, _M, _H), lambda i: (0, i, 0)),
            pl.BlockSpec((1, _H), lambda i: (0, 0)),
        ],
        out_specs=pl.BlockSpec((_M, 16), lambda i: (i, 0)),
        out_shape=jax.ShapeDtypeStruct((_NPAD, 16), jnp.float32),
    )(hist, acc, b2)


def _tc_pool_body(colors_ref, wp_ref, bp_ref, wo_ref, bo_ref, sat_ref):
    colors = colors_ref[...]
    neg = jnp.float32(-1e30)
    gate = jnp.sum(colors * wp_ref[...], axis=1, keepdims=True) + bp_ref[0, 0]
    row = lax.broadcasted_iota(jnp.int32, (_NPAD, 1), 0)
    rowmask = row < _N
    glog = jnp.where(rowmask, gate, neg)
    gm = jnp.max(glog, axis=0, keepdims=True)
    ge = jnp.where(rowmask, jnp.exp(glog - gm), 0.0)
    gw = ge / jnp.sum(ge, axis=0, keepdims=True)
    readout = jnp.sum(gw * colors, axis=0, keepdims=True)
    logit = jnp.sum(readout * wo_ref[...]) + bo_ref[0, 0]
    sat = 1.0 / (1.0 + jnp.exp(-logit))
    sat_ref[...] = jnp.reshape(sat, (1, 1))


def _tc_pool(colors, wp, bp, wo, bo):
    return pl.pallas_call(
        _tc_pool_body,
        out_shape=jax.ShapeDtypeStruct((1, 1), jnp.float32),
    )(colors, wp, bp, wo, bo)


def kernel(x, edge_index, W0, b0, W1, b1, W2, b2, Wp, bp, Wo, bo):
    f32 = jnp.float32
    src = edge_index[0]
    dst = edge_index[1]
    pad = _N + (jnp.arange(_EPAD - _E, dtype=jnp.int32) % (_NPAD - _N))
    src_slabs = jnp.concatenate([src, pad]).reshape(_NW, _CH, _K)
    dst_slabs = jnp.concatenate([dst, pad]).reshape(_NW, _CH, _K)

    x_pad = jnp.zeros((_NPAD, _H), f32).at[:_N].set(x)
    zeros128 = jnp.zeros((_ROWS, _H), f32)
    zeros16 = jnp.zeros((_ROWS, 16), f32)
    ones16 = jnp.ones((_K, 16), f32)

    b0r = b0.reshape(1, _H)
    b1r = b1.reshape(1, _H)
    W2p = jnp.zeros((_H, _H), f32).at[:, :3].set(W2)
    b2r = jnp.zeros((1, _H), f32).at[0, :3].set(b2)
    wp_row = jnp.zeros((1, 16), f32).at[0, :3].set(Wp[:, 0])
    wo_row = jnp.zeros((1, 16), f32).at[0, :3].set(Wo[:, 0])
    bpr = bp.reshape(1, 1)
    bor = bo.reshape(1, 1)

    src_flat = src_slabs.reshape(_NW * _CH, _K)
    dst_flat = dst_slabs.reshape(_NW * _CH, _K)

    hist = _sc_degree(src_slabs, dst_slabs, ones16, zeros16)

    z0 = _tc_z0(hist, x_pad, W0)
    acc1 = _sc_spmm(z0, src_flat, dst_flat, zeros128)
    z1 = _tc_mid(hist, acc1, b0r, W1)
    acc2 = _sc_spmm(z1, src_flat, dst_flat, zeros128)
    z2 = _tc_mid(hist, acc2, b1r, W2p)
    acc3 = _sc_spmm(z2, src_flat, dst_flat, zeros128)
    colors_pad = _tc_colors(hist, acc3, b2r)
    sat = _tc_pool(colors_pad, wp_row, bpr, wo_row, bor)

    return colors_pad[:_N, :3], sat[0, 0]


# speedup vs baseline: 4.3486x; 1.0010x over previous
import functools

import jax
import jax.numpy as jnp
from jax import lax
from jax.experimental import pallas as pl
from jax.experimental.pallas import tpu as pltpu
from jax.experimental.pallas import tpu_sc as plsc

_N = 10000
_E = 320000
_H = 128
_NC = 2
_NS = 16
_NW = _NC * _NS
_K = 128
_NPAD = 10240
_ROWS = _NPAD // _NS
_CH = 80
_EPAD = _CH * _NW * _K


def _sc_mesh():
    return plsc.VectorSubcoreMesh(core_axis_name="c", subcore_axis_name="s")


def _sc_degree_body(srcs, dsts, ones_h, zeros_h, out,
                    src_slab, dst_slab, ones_v, acc, sem):
    c = lax.axis_index("c")
    s = lax.axis_index("s")
    wid = s * _NC + c
    r0 = s * _ROWS

    pltpu.sync_copy(zeros_h, acc.at[pl.ds(r0, _ROWS)])
    pltpu.sync_copy(ones_h, ones_v)
    pltpu.sync_copy(srcs.at[pl.ds(wid * _CH, _CH)], src_slab)
    pltpu.sync_copy(dsts.at[pl.ds(wid * _CH, _CH)], dst_slab)
    plsc.subcore_barrier()

    def step_s(j, carry):
        pltpu.sync_copy(ones_v, acc.at[src_slab.at[j]], add=True)
        return carry

    lax.fori_loop(0, _CH, step_s, 0)
    plsc.subcore_barrier()
    pltpu.sync_copy(acc.at[pl.ds(r0, _ROWS)], out.at[c, 0, pl.ds(r0, _ROWS)])
    pltpu.sync_copy(zeros_h, acc.at[pl.ds(r0, _ROWS)])
    plsc.subcore_barrier()

    def step_d(j, carry):
        pltpu.sync_copy(ones_v, acc.at[dst_slab.at[j]], add=True)
        return carry

    lax.fori_loop(0, _CH, step_d, 0)
    plsc.subcore_barrier()
    pltpu.sync_copy(acc.at[pl.ds(r0, _ROWS)], out.at[c, 1, pl.ds(r0, _ROWS)])


def _sc_degree(src_slabs, dst_slabs, ones_h, zeros_h):
    kfn = pl.kernel(
        _sc_degree_body,
        mesh=_sc_mesh(),
        out_type=jax.ShapeDtypeStruct((_NC, 2, _NPAD, 16), jnp.float32),
        scratch_types=[
            pltpu.VMEM((_CH, _K), jnp.int32),
            pltpu.VMEM((_CH, _K), jnp.int32),
            pltpu.VMEM((_K, 16), jnp.float32),
            pltpu.VMEM_SHARED((_NPAD, 16), jnp.float32),
            pltpu.SemaphoreType.DMA,
        ],
        compiler_params=pltpu.CompilerParams(use_tc_tiling_on_sc=False),
    )
    return kfn(src_slabs, dst_slabs, ones_h, zeros_h)


_CH0 = 80
_CH1 = 80
_BLK = 16


def _sc_spmm_body(z, srcs, dsts, zeros_h, out,
                  src_slab, dst_slab, rows0, rows1, acc, gsem0, gsem1):
    c = lax.axis_index("c")
    s = lax.axis_index("s")
    r0 = s * _ROWS
    cnt = _CH0 + c * (_CH1 - _CH0)
    start = c * (_NS * _CH0) + s * cnt

    pltpu.sync_copy(zeros_h, acc.at[pl.ds(r0, _ROWS)])
    plsc.subcore_barrier()

    def blk(b, carry):
        base = start + b * _BLK
        pltpu.sync_copy(srcs.at[pl.ds(base, _BLK)], src_slab)
        pltpu.sync_copy(dsts.at[pl.ds(base, _BLK)], dst_slab)
        pltpu.async_copy(z.at[src_slab.at[0]], rows0, gsem0)

        def step(jj, carry2):
            j0 = 2 * jj
            j1 = j0 + 1
            pltpu.make_async_copy(z.at[src_slab.at[j0]], rows0, gsem0).wait()
            pltpu.async_copy(z.at[src_slab.at[j1]], rows1, gsem1)
            pltpu.sync_copy(rows0, acc.at[dst_slab.at[j0]], add=True)
            pltpu.make_async_copy(z.at[src_slab.at[j1]], rows1, gsem1).wait()

            @pl.when(j1 + 1 < _BLK)
            def _():
                pltpu.async_copy(z.at[src_slab.at[j1 + 1]], rows0, gsem0)

            pltpu.sync_copy(rows1, acc.at[dst_slab.at[j1]], add=True)
            return carry2

        lax.fori_loop(0, _BLK // 2, step, 0)
        return carry

    lax.fori_loop(0, cnt // _BLK, blk, 0)
    plsc.subcore_barrier()
    pltpu.sync_copy(acc.at[pl.ds(r0, _ROWS)], out.at[c, pl.ds(r0, _ROWS)])


def _sc_spmm(z, src_flat, dst_flat, zeros_h):
    w = z.shape[1]
    kfn = pl.kernel(
        _sc_spmm_body,
        mesh=_sc_mesh(),
        out_type=jax.ShapeDtypeStruct((_NC, _NPAD, w), jnp.float32),
        scratch_types=[
            pltpu.VMEM((_BLK, _K), jnp.int32),
            pltpu.VMEM((_BLK, _K), jnp.int32),
            pltpu.VMEM((_K, w), jnp.float32),
            pltpu.VMEM((_K, w), jnp.float32),
            pltpu.VMEM_SHARED((_NPAD, w), jnp.float32),
            pltpu.SemaphoreType.DMA,
            pltpu.SemaphoreType.DMA,
        ],
    )
    return kfn(z, src_flat, dst_flat, zeros_h)


_M = 512


def _norms_from_hist(h4):
    deg_s = h4[0, 0, :, 0:1] + h4[1, 0, :, 0:1]
    deg_d = h4[0, 1, :, 0:1] + h4[1, 1, :, 0:1]
    norm_s = lax.rsqrt(jnp.maximum(deg_s, 1.0))
    norm_d = lax.rsqrt(jnp.maximum(deg_d, 1.0))
    return norm_s, norm_d


def _tc_z0_body(hist_ref, x_ref, w_ref, o_ref):
    norm_s, _ = _norms_from_hist(hist_ref[...])
    o_ref[...] = jnp.dot(x_ref[...] * norm_s, w_ref[...],
                         preferred_element_type=jnp.float32)


def _tc_z0(hist, x, w):
    return pl.pallas_call(
        _tc_z0_body,
        grid=(_NPAD // _M,),
        in_specs=[
            pl.BlockSpec((_NC, 2, _M, 16), lambda i: (0, 0, i, 0)),
            pl.BlockSpec((_M, _H), lambda i: (i, 0)),
            pl.BlockSpec((_H, _H), lambda i: (0, 0)),
        ],
        out_specs=pl.BlockSpec((_M, _H), lambda i: (i, 0)),
        out_shape=jax.ShapeDtypeStruct((_NPAD, _H), jnp.float32),
    )(hist, x, w)


def _tc_mid_body(hist_ref, acc_ref, b_ref, w_ref, o_ref):
    norm_s, norm_d = _norms_from_hist(hist_ref[...])
    a = acc_ref[...]
    y = (a[0] + a[1]) * norm_d + b_ref[...]
    h = jnp.maximum(y, 0.0)
    o_ref[...] = jnp.dot(h * norm_s, w_ref[...],
                         preferred_element_type=jnp.float32)


def _tc_mid(hist, acc, b, w):
    wo = w.shape[1]
    return pl.pallas_call(
        _tc_mid_body,
        grid=(_NPAD // _M,),
        in_specs=[
            pl.BlockSpec((_NC, 2, _M, 16), lambda i: (0, 0, i, 0)),
            pl.BlockSpec((_NC, _M, _H), lambda i: (0, i, 0)),
            pl.BlockSpec((1, _H), lambda i: (0, 0)),
            pl.BlockSpec((_H, wo), lambda i: (0, 0)),
        ],
        out_specs=pl.BlockSpec((_M, wo), lambda i: (i, 0)),
        out_shape=jax.ShapeDtypeStruct((_NPAD, wo), jnp.float32),
    )(hist, acc, b, w)


def _tc_colors_body(hist_ref, acc_ref, b2_ref, colors_ref):
    h4 = hist_ref[...]
    deg_d = h4[0, 1, :, 0:1] + h4[1, 1, :, 0:1]
    norm_d = lax.rsqrt(jnp.maximum(deg_d, 1.0))
    a = acc_ref[...]
    y = (a[0] + a[1]) * norm_d + b2_ref[...]
    neg = jnp.float32(-1e30)
    col = lax.broadcasted_iota(jnp.int32, (_M, _H), 1)
    colmask = col < 3
    logits = jnp.where(colmask, y, neg)
    m = jnp.max(logits, axis=1, keepdims=True)
    e = jnp.where(colmask, jnp.exp(logits - m), 0.0)
    colors = e / jnp.sum(e, axis=1, keepdims=True)
    colors_ref[...] = colors[:, :16]


def _tc_colors(hist, acc, b2):
    return pl.pallas_call(
        _tc_colors_body,
        grid=(_NPAD // _M,),
        in_specs=[
            pl.BlockSpec((_NC, 2, _M, 16), lambda i: (0, 0, i, 0)),
            pl.BlockSpec((_NC, _M, _H), lambda i: (0, i, 0)),
            pl.BlockSpec((1, _H), lambda i: (0, 0)),
        ],
        out_specs=pl.BlockSpec((_M, 16), lambda i: (i, 0)),
        out_shape=jax.ShapeDtypeStruct((_NPAD, 16), jnp.float32),
    )(hist, acc, b2)


def _tc_pool_body(colors_ref, wp_ref, bp_ref, wo_ref, bo_ref, sat_ref):
    colors = colors_ref[...]
    neg = jnp.float32(-1e30)
    gate = jnp.sum(colors * wp_ref[...], axis=1, keepdims=True) + bp_ref[0, 0]
    row = lax.broadcasted_iota(jnp.int32, (_NPAD, 1), 0)
    rowmask = row < _N
    glog = jnp.where(rowmask, gate, neg)
    gm = jnp.max(glog, axis=0, keepdims=True)
    ge = jnp.where(rowmask, jnp.exp(glog - gm), 0.0)
    gw = ge / jnp.sum(ge, axis=0, keepdims=True)
    readout = jnp.sum(gw * colors, axis=0, keepdims=True)
    logit = jnp.sum(readout * wo_ref[...]) + bo_ref[0, 0]
    sat = 1.0 / (1.0 + jnp.exp(-logit))
    sat_ref[...] = jnp.reshape(sat, (1, 1))


def _tc_pool(colors, wp, bp, wo, bo):
    return pl.pallas_call(
        _tc_pool_body,
        out_shape=jax.ShapeDtypeStruct((1, 1), jnp.float32),
    )(colors, wp, bp, wo, bo)


def kernel(x, edge_index, W0, b0, W1, b1, W2, b2, Wp, bp, Wo, bo):
    f32 = jnp.float32
    src = edge_index[0]
    dst = edge_index[1]
    pad = _N + (jnp.arange(_EPAD - _E, dtype=jnp.int32) % (_NPAD - _N))
    src_flat = jnp.concatenate([src, pad]).reshape(_NW * _CH, _K)
    dst_flat = jnp.concatenate([dst, pad]).reshape(_NW * _CH, _K)

    x_pad = jnp.zeros((_NPAD, _H), f32).at[:_N].set(x)
    zeros128 = jnp.zeros((_ROWS, _H), f32)
    zeros16 = jnp.zeros((_ROWS, 16), f32)
    ones16 = jnp.ones((_K, 16), f32)

    b0r = b0.reshape(1, _H)
    b1r = b1.reshape(1, _H)
    W2p = jnp.zeros((_H, _H), f32).at[:, :3].set(W2)
    b2r = jnp.zeros((1, _H), f32).at[0, :3].set(b2)
    wp_row = jnp.zeros((1, 16), f32).at[0, :3].set(Wp[:, 0])
    wo_row = jnp.zeros((1, 16), f32).at[0, :3].set(Wo[:, 0])
    bpr = bp.reshape(1, 1)
    bor = bo.reshape(1, 1)

    hist = _sc_degree(src_flat, dst_flat, ones16, zeros16)

    z0 = _tc_z0(hist, x_pad, W0)
    acc1 = _sc_spmm(z0, src_flat, dst_flat, zeros128)
    z1 = _tc_mid(hist, acc1, b0r, W1)
    acc2 = _sc_spmm(z1, src_flat, dst_flat, zeros128)
    z2 = _tc_mid(hist, acc2, b1r, W2p)
    acc3 = _sc_spmm(z2, src_flat, dst_flat, zeros128)
    colors_pad = _tc_colors(hist, acc3, b2r)
    sat = _tc_pool(colors_pad, wp_row, bpr, wo_row, bor)

    return colors_pad[:_N, :3], sat[0, 0]
